# baseline (speedup 1.0000x reference)
import jax
import jax.numpy as jnp
from jax.experimental import pallas as pl
from jax.experimental.pallas import tpu as pltpu

N = 8192
D = 2048
E = 64
K = 8
CAP = float(int(1.25 * N / E))

EPAD = 128
TILE = 1024
NBLK = N // TILE
NEG = -1e30


def _router_body(x_ref, wrt_ref, br_ref,
                 out_ref, pcounts_ref, pconf_ref):
    out_ref[...] = jnp.zeros_like(out_ref)

    logits = jnp.dot(x_ref[...], wrt_ref[...],
                     preferred_element_type=jnp.float32) + br_ref[...]

    m = jnp.max(logits, axis=1, keepdims=True)
    pconf_ref[...] = jnp.sum(m).reshape(1, 1, 1)
    vals = logits
    for _ in range(K - 1):
        vals = jnp.where(vals >= m, jnp.float32(NEG), vals)
        m = jnp.max(vals, axis=1, keepdims=True)
    member = (logits >= m).astype(jnp.float32)
    pcounts_ref[...] = jnp.sum(member, axis=0).reshape(1, 1, EPAD)


def _stats_body(pcounts_ref, pconf_ref, loss_ref, dist_ref, conf_ref):
    counts = jnp.sum(pcounts_ref[...], axis=(0, 1)).reshape(1, EPAD)
    load = jnp.minimum(counts, jnp.float32(CAP))
    s = jnp.sum(load)
    dist = load / (s + jnp.float32(1e-8))
    dist_ref[...] = dist
    loss_ref[...] = jnp.sum(dist * jnp.log(dist + jnp.float32(1e-8))).reshape(1, 1)
    conf_ref[...] = jnp.sum(pconf_ref[...]).reshape(1, 1) * jnp.float32(1.0 / N)


def kernel(x, Wr, br, W1, b1, W2, b2):
    del W1, b1, W2, b2
    wrt = jnp.pad(Wr.T, ((0, 0), (0, EPAD - E)))
    brp = jnp.pad(br.reshape(1, E), ((0, 0), (0, EPAD - E)),
                  constant_values=NEG)

    out, pcounts, pconf = pl.pallas_call(
        _router_body,
        grid=(NBLK,),
        in_specs=[
            pl.BlockSpec((TILE, D), lambda i: (i, 0)),
            pl.BlockSpec((D, EPAD), lambda i: (0, 0)),
            pl.BlockSpec((1, EPAD), lambda i: (0, 0)),
        ],
        out_specs=[
            pl.BlockSpec((TILE, D), lambda i: (i, 0)),
            pl.BlockSpec((1, 1, EPAD), lambda i: (i, 0, 0)),
            pl.BlockSpec((1, 1, 1), lambda i: (i, 0, 0)),
        ],
        out_shape=[
            jax.ShapeDtypeStruct((N, D), jnp.float32),
            jax.ShapeDtypeStruct((NBLK, 1, EPAD), jnp.float32),
            jax.ShapeDtypeStruct((NBLK, 1, 1), jnp.float32),
        ],
        compiler_params=pltpu.CompilerParams(
            dimension_semantics=("parallel",)),
    )(x, wrt, brp)

    loss, dist, conf = pl.pallas_call(
        _stats_body,
        out_shape=[
            jax.ShapeDtypeStruct((1, 1), jnp.float32),
            jax.ShapeDtypeStruct((1, EPAD), jnp.float32),
            jax.ShapeDtypeStruct((1, 1), jnp.float32),
        ],
    )(pcounts, pconf)

    return (out,
            loss.reshape(()),
            dist[0, :E],
            conf.reshape(()))
